# baseline (speedup 1.0000x reference)
import functools

import jax
import jax.numpy as jnp
from jax import lax
from jax.experimental import pallas as pl
from jax.experimental.pallas import tpu as pltpu
from jax.experimental.pallas import tpu_sc as plsc

_N, _E, _D = 10000, 160000, 256
_NW = 32
_RPS = 320
_NPAD = _NW * _RPS
_CH = 32
_NCHUNKS = _E // _CH
_NG = _D // 16


def _tc_body(x_ref, w_ref, a_ref, h_ref, s_ref):
    hb = jnp.dot(x_ref[...], w_ref[...], preferred_element_type=jnp.float32)
    h_ref[...] = hb
    s_ref[...] = jnp.dot(hb, a_ref[...], preferred_element_type=jnp.float32)


def _tc_transform(x, w, a):
    m = 400
    grid = _N // m
    return pl.pallas_call(
        _tc_body,
        grid=(grid,),
        in_specs=[
            pl.BlockSpec((m, _D), lambda i: (i, 0)),
            pl.BlockSpec((_D, _D), lambda i: (0, 0)),
            pl.BlockSpec((_D, 2), lambda i: (0, 0)),
        ],
        out_specs=[
            pl.BlockSpec((m, _D), lambda i: (i, 0)),
            pl.BlockSpec((m, 2), lambda i: (i, 0)),
        ],
        out_shape=[
            jax.ShapeDtypeStruct((_N, _D), jnp.float32),
            jax.ShapeDtypeStruct((_N, 2), jnp.float32),
        ],
    )(x, w, a)


def _sc_body(h_hbm, spair_hbm, src_hbm, dst_hbm, bnd_hbm, out_hbm,
             sp_v, bnd_v, srcv, dstv, srcv2, dstv2, rows_v, rows_v2,
             acc_v, sacc_v, sem, sem2):
    cid = lax.axis_index("c")
    sid = lax.axis_index("s")
    w = cid * 16 + sid
    lo = w * _RPS
    iota = lax.iota(jnp.int32, 16)
    zf = jnp.zeros((16,), jnp.float32)

    pltpu.sync_copy(spair_hbm, sp_v)
    pltpu.sync_copy(bnd_hbm, bnd_v)

    def _zero(r, c):
        for q in range(_NG):
            acc_v[r, pl.ds(16 * q, 16)] = zf
        sacc_v[r, pl.ds(0, 16)] = zf
        return c
    lax.fori_loop(0, _RPS, _zero, 0)

    bndg = plsc.load_gather(bnd_v, [jnp.minimum(w + iota, 39)])
    c0 = (bndg >> 5)[0]
    c1 = jnp.minimum((bndg + (_CH - 1)) >> 5, _NCHUNKS)[1]

    zi = jnp.zeros((16,), jnp.int32)

    def _scores(sv, dv):
        scs, locs = [], []
        for j in range(_CH // 16):
            s16 = sv[pl.ds(16 * j, 16)]
            d16 = dv[pl.ds(16 * j, 16)]
            g1 = plsc.load_gather(sp_v, [s16 * 2])
            g2 = plsc.load_gather(sp_v, [d16 * 2 + 1])
            x = g1 + g2
            x = jnp.where(x >= 0.0, x, 0.2 * x)
            x = jnp.clip(x, -2.0, 2.0)
            sc = jnp.exp(x)
            owned = (s16 >= lo) & (s16 < lo + _RPS)
            scs.append(jnp.where(owned, sc, 0.0))
            locs.append(jnp.clip(s16 - lo, 0, _RPS - 1))
        return scs, locs

    def _accum(scs, locs, rv):
        for j in range(_CH // 16):
            sc16, loc16 = scs[j], locs[j]
            plsc.addupdate_scatter(sacc_v, [loc16, iota], sc16)
            for e2 in range(16):
                e = 16 * j + e2
                se = sc16[e2]
                rowv = zi + loc16[e2]
                for q in range(_NG):
                    plsc.addupdate_scatter(
                        acc_v, [rowv, 16 * q + iota],
                        rv[e, pl.ds(16 * q, 16)] * se)

    def _load_edges(ci, sv, dv):
        e0 = ci * _CH
        pltpu.sync_copy(src_hbm.at[pl.ds(e0, _CH)], sv)
        pltpu.sync_copy(dst_hbm.at[pl.ds(e0, _CH)], dv)

    @pl.when(c0 < c1)
    def _():
        _load_edges(c0, srcv, dstv)
        pltpu.async_copy(h_hbm.at[dstv], rows_v, sem)

    def _pair(k, c):
        i = c0 + 2 * k

        @pl.when(i + 1 < c1)
        def _():
            _load_edges(i + 1, srcv2, dstv2)
            pltpu.async_copy(h_hbm.at[dstv2], rows_v2, sem2)
        scs, locs = _scores(srcv, dstv)
        pltpu.make_async_copy(h_hbm.at[dstv], rows_v, sem).wait()
        _accum(scs, locs, rows_v)

        @pl.when(i + 1 < c1)
        def _():
            @pl.when(i + 2 < c1)
            def _():
                _load_edges(i + 2, srcv, dstv)
                pltpu.async_copy(h_hbm.at[dstv], rows_v, sem)
            scs2, locs2 = _scores(srcv2, dstv2)
            pltpu.make_async_copy(h_hbm.at[dstv2], rows_v2, sem2).wait()
            _accum(scs2, locs2, rows_v2)
        return c
    lax.fori_loop(0, (c1 - c0 + 1) >> 1, _pair, 0)

    def _norm(r, c):
        ssr = sacc_v[r, pl.ds(0, 16)]
        ssum = jnp.sum(ssr)
        bc = zf + ssum
        invv = 1.0 / jnp.where(bc == 0.0, 1.0, bc)
        inv = invv[0]
        for q in range(_NG):
            acc_v[r, pl.ds(16 * q, 16)] = acc_v[r, pl.ds(16 * q, 16)] * inv
        return c
    lax.fori_loop(0, _RPS, _norm, 0)
    pltpu.sync_copy(acc_v, out_hbm.at[pl.ds(lo, _RPS)])


_sc_edge = functools.partial(
    pl.kernel,
    out_type=jax.ShapeDtypeStruct((_NPAD, _D), jnp.float32),
    mesh=plsc.VectorSubcoreMesh(core_axis_name="c", subcore_axis_name="s"),
    compiler_params=pltpu.CompilerParams(needs_layout_passes=False,
                                         use_tc_tiling_on_sc=False),
    scratch_types=[
        pltpu.VMEM((2 * _N,), jnp.float32),
        pltpu.VMEM((40,), jnp.int32),
        pltpu.VMEM((_CH,), jnp.int32),
        pltpu.VMEM((_CH,), jnp.int32),
        pltpu.VMEM((_CH,), jnp.int32),
        pltpu.VMEM((_CH,), jnp.int32),
        pltpu.VMEM((_CH, _D), jnp.float32),
        pltpu.VMEM((_CH, _D), jnp.float32),
        pltpu.VMEM((_RPS, _D), jnp.float32),
        pltpu.VMEM((_RPS, 16), jnp.float32),
        pltpu.SemaphoreType.DMA,
        pltpu.SemaphoreType.DMA,
    ],
)(_sc_body)


def kernel(node_states, edges, kernel, kernel_attention):
    src = edges[:, 0].astype(jnp.int32)
    dst = edges[:, 1].astype(jnp.int32)
    u = kernel.shape[1]
    a = jnp.stack([kernel_attention[:u, 0], kernel_attention[u:, 0]], axis=1)
    h, s_pair = _tc_transform(node_states.astype(jnp.float32),
                              kernel.astype(jnp.float32), a)
    bnd = jnp.searchsorted(src, jnp.arange(33, dtype=jnp.int32) * _RPS)
    bnd = jnp.concatenate([bnd.astype(jnp.int32),
                           jnp.zeros((7,), jnp.int32)])
    out_pad = _sc_edge(h, s_pair.reshape(-1), src, dst, bnd)
    return out_pad[:_N]


# speedup vs baseline: 3.4539x; 3.4539x over previous
import functools

import jax
import jax.numpy as jnp
from jax import lax
from jax.experimental import pallas as pl
from jax.experimental.pallas import tpu as pltpu
from jax.experimental.pallas import tpu_sc as plsc

_N, _E, _D = 10000, 160000, 256
_NW = 32
_RPS = 320
_NPAD = _NW * _RPS
_CH = 32
_NCHUNKS = _E // _CH
_NG = _D // 16
_SBC = 64
_SBE = _SBC * _CH


def _tc_body(x_ref, w_ref, a_ref, h_ref, s_ref):
    hb = jnp.dot(x_ref[...], w_ref[...], preferred_element_type=jnp.float32)
    h_ref[...] = hb
    s_ref[...] = jnp.dot(hb, a_ref[...], preferred_element_type=jnp.float32)


def _tc_transform(x, w, a):
    m = 400
    grid = _N // m
    return pl.pallas_call(
        _tc_body,
        grid=(grid,),
        in_specs=[
            pl.BlockSpec((m, _D), lambda i: (i, 0)),
            pl.BlockSpec((_D, _D), lambda i: (0, 0)),
            pl.BlockSpec((_D, 2), lambda i: (0, 0)),
        ],
        out_specs=[
            pl.BlockSpec((m, _D), lambda i: (i, 0)),
            pl.BlockSpec((m, 2), lambda i: (i, 0)),
        ],
        out_shape=[
            jax.ShapeDtypeStruct((_N, _D), jnp.float32),
            jax.ShapeDtypeStruct((_N, 2), jnp.float32),
        ],
    )(x, w, a)


def _sc_body(h_hbm, spair_hbm, src_hbm, dst_hbm, bnd_hbm, out_hbm,
             sp_v, bnd_v, sbsrc, sbdst, srcv, dstv, srcv2, dstv2,
             rows_v, rows_v2, scb_v, locb_v, acc_v, sacc_v, sem, sem2):
    cid = lax.axis_index("c")
    sid = lax.axis_index("s")
    w = cid * 16 + sid
    lo = w * _RPS
    iota = lax.iota(jnp.int32, 16)
    zf = jnp.zeros((16,), jnp.float32)

    pltpu.sync_copy(spair_hbm, sp_v.at[pl.ds(0, 2 * _N)])
    pltpu.sync_copy(bnd_hbm, bnd_v)

    def _zero(r, c):
        for q in range(_NG):
            acc_v[r, pl.ds(16 * q, 16)] = zf
        sacc_v[r, pl.ds(0, 16)] = zf
        return c
    lax.fori_loop(0, _RPS, _zero, 0)

    bndg = plsc.load_gather(bnd_v, [jnp.minimum(w + iota, 39)])
    c0 = (bndg >> 5)[0]
    c1 = jnp.minimum((bndg + (_CH - 1)) >> 5, _NCHUNKS)[1]

    zi = jnp.zeros((16,), jnp.int32)

    def _scores(sv, dv):
        for j in range(_CH // 16):
            s16 = sv[pl.ds(16 * j, 16)]
            d16 = dv[pl.ds(16 * j, 16)]
            g1 = plsc.load_gather(sp_v, [s16 * 2])
            g2 = plsc.load_gather(sp_v, [d16 * 2 + 1])
            x = g1 + g2
            x = jnp.where(x >= 0.0, x, 0.2 * x)
            x = jnp.clip(x, -2.0, 2.0)
            sc = jnp.exp(x)
            owned = (s16 >= lo) & (s16 < lo + _RPS)
            sc16 = jnp.where(owned, sc, 0.0)
            loc16 = jnp.clip(s16 - lo, 0, _RPS - 1)
            plsc.addupdate_scatter(sacc_v, [loc16, iota], sc16)
            for e2 in range(16):
                scb_v[16 * j + e2, pl.ds(0, 16)] = zf + sc16[e2]
                locb_v[16 * j + e2, pl.ds(0, 16)] = zi + loc16[e2]

    def _accum(rv):
        @plsc.parallel_loop(0, _CH, unroll=4)
        def _edge(e):
            sev = scb_v[e, pl.ds(0, 16)]
            rowv = locb_v[e, pl.ds(0, 16)]
            for q in range(_NG):
                plsc.addupdate_scatter(
                    acc_v, [rowv, 16 * q + iota],
                    rv[e, pl.ds(16 * q, 16)] * sev)

    def _load_edges(cc, sv, dv):
        for j in range(_CH // 16):
            sv[pl.ds(16 * j, 16)] = sbsrc[pl.ds(_CH * cc + 16 * j, 16)]
            dv[pl.ds(16 * j, 16)] = sbdst[pl.ds(_CH * cc + 16 * j, 16)]

    def _sblock(sb, c):
        cb = c0 + _SBC * sb
        c1s = jnp.minimum(c1, cb + _SBC)
        pltpu.sync_copy(src_hbm.at[pl.ds(cb * _CH, _SBE)], sbsrc)
        pltpu.sync_copy(dst_hbm.at[pl.ds(cb * _CH, _SBE)], sbdst)

        @pl.when(cb < c1s)
        def _():
            _load_edges(0, srcv, dstv)
            pltpu.async_copy(h_hbm.at[dstv], rows_v, sem)

        def _pair(k, cc):
            i = cb + 2 * k

            @pl.when(i + 1 < c1s)
            def _():
                _load_edges(2 * k + 1, srcv2, dstv2)
                pltpu.async_copy(h_hbm.at[dstv2], rows_v2, sem2)
            _scores(srcv, dstv)
            pltpu.make_async_copy(h_hbm.at[dstv], rows_v, sem).wait()
            _accum(rows_v)

            @pl.when(i + 1 < c1s)
            def _():
                @pl.when(i + 2 < c1s)
                def _():
                    _load_edges(2 * k + 2, srcv, dstv)
                    pltpu.async_copy(h_hbm.at[dstv], rows_v, sem)
                _scores(srcv2, dstv2)
                pltpu.make_async_copy(h_hbm.at[dstv2], rows_v2, sem2).wait()
                _accum(rows_v2)
            return cc
        lax.fori_loop(0, (c1s - cb + 1) >> 1, _pair, 0)
        return c
    nsb = (c1 - c0 + _SBC - 1) >> 6
    lax.fori_loop(0, nsb, _sblock, 0)

    def _norm(r, c):
        ssr = sacc_v[r, pl.ds(0, 16)]
        ssum = jnp.sum(ssr)
        bc = zf + ssum
        invv = 1.0 / jnp.where(bc == 0.0, 1.0, bc)
        inv = invv[0]
        for q in range(_NG):
            acc_v[r, pl.ds(16 * q, 16)] = acc_v[r, pl.ds(16 * q, 16)] * inv
        return c
    lax.fori_loop(0, _RPS, _norm, 0)
    pltpu.sync_copy(acc_v, out_hbm.at[pl.ds(lo, _RPS)])


_sc_edge = functools.partial(
    pl.kernel,
    out_type=jax.ShapeDtypeStruct((_NPAD, _D), jnp.float32),
    mesh=plsc.VectorSubcoreMesh(core_axis_name="c", subcore_axis_name="s"),
    compiler_params=pltpu.CompilerParams(needs_layout_passes=False,
                                         use_tc_tiling_on_sc=False),
    scratch_types=[
        pltpu.VMEM((2 * _NPAD + 8,), jnp.float32),
        pltpu.VMEM((40,), jnp.int32),
        pltpu.VMEM((_SBE,), jnp.int32),
        pltpu.VMEM((_SBE,), jnp.int32),
        pltpu.VMEM((_CH,), jnp.int32),
        pltpu.VMEM((_CH,), jnp.int32),
        pltpu.VMEM((_CH,), jnp.int32),
        pltpu.VMEM((_CH,), jnp.int32),
        pltpu.VMEM((_CH, _D), jnp.float32),
        pltpu.VMEM((_CH, _D), jnp.float32),
        pltpu.VMEM((_CH, 16), jnp.float32),
        pltpu.VMEM((_CH, 16), jnp.int32),
        pltpu.VMEM((_RPS, _D), jnp.float32),
        pltpu.VMEM((_RPS, 16), jnp.float32),
        pltpu.SemaphoreType.DMA,
        pltpu.SemaphoreType.DMA,
    ],
)(_sc_body)


def kernel(node_states, edges, kernel, kernel_attention):
    src = edges[:, 0].astype(jnp.int32)
    dst = edges[:, 1].astype(jnp.int32)
    u = kernel.shape[1]
    a = jnp.stack([kernel_attention[:u, 0], kernel_attention[u:, 0]], axis=1)
    h, s_pair = _tc_transform(node_states.astype(jnp.float32),
                              kernel.astype(jnp.float32), a)
    bnd = jnp.searchsorted(src, jnp.arange(33, dtype=jnp.int32) * _RPS)
    bnd = jnp.concatenate([bnd.astype(jnp.int32),
                           jnp.zeros((7,), jnp.int32)])
    src_p = jnp.concatenate([src, jnp.full((_SBE,), _NPAD, jnp.int32)])
    dst_p = jnp.concatenate([dst, jnp.zeros((_SBE,), jnp.int32)])
    out_pad = _sc_edge(h, s_pair.reshape(-1), src_p, dst_p, bnd)
    return out_pad[:_N]
